# baseline (speedup 1.0000x reference)
import functools

import jax
import jax.numpy as jnp
from jax import lax
from jax.experimental import pallas as pl
from jax.experimental.pallas import tpu as pltpu
from jax.experimental.pallas import tpu_sc as plsc

L = 16
N_ROIS = 512
NC, NS = 2, 16
NW = NC * NS
ROIS_PER_W = N_ROIS // NW
C0, H0, CR0 = 192, 64, 256
C1, H1, CR1 = 384, 32, 384
OUT_PIX = 49
OFF1 = C0 * OUT_PIX
OUT_COLS = (C0 + C1) * OUT_PIX


def _tp(x_ref, o_ref):
    t = jnp.transpose(x_ref[...], (0, 2, 3, 1))
    pad = o_ref.shape[-1] - t.shape[-1]
    if pad:
        t = jnp.concatenate(
            [t, jnp.zeros(t.shape[:-1] + (pad,), t.dtype)], axis=-1)
    o_ref[...] = t


def _tables_body(x0_ref, x1_ref, o0_ref, o1_ref):
    _tp(x0_ref, o0_ref)
    _tp(x1_ref, o1_ref)


def _to_tables(feat0, feat1):
    t0, t1 = pl.pallas_call(
        _tables_body,
        grid=(2, 2),
        in_specs=[pl.BlockSpec((1, C0, H0 // 2, H0), lambda b, h: (b, 0, h, 0)),
                  pl.BlockSpec((1, C1, H1 // 2, H1), lambda b, h: (b, 0, h, 0))],
        out_specs=[pl.BlockSpec((1, H0 // 2, H0, CR0),
                                lambda b, h: (b, h, 0, 0)),
                   pl.BlockSpec((1, H1 // 2, H1, CR1),
                                lambda b, h: (b, h, 0, 0))],
        out_shape=[jax.ShapeDtypeStruct((2, H0, H0, CR0), jnp.float32),
                   jax.ShapeDtypeStruct((2, H1, H1, CR1), jnp.float32)],
    )(feat0, feat1)
    return t0.reshape(2 * H0 * H0, CR0), t1.reshape(2 * H1 * H1, CR1)


def _relayout_body(x_ref, o_ref):
    o_ref[...] = jnp.transpose(x_ref[...], (2, 1, 0))


def _to_output(flat):
    v = flat.reshape(N_ROIS, C0 + C1, OUT_PIX)
    res = pl.pallas_call(
        _relayout_body,
        grid=(2, 9),
        in_specs=[pl.BlockSpec((256, 64, OUT_PIX), lambda nb, cb: (nb, cb, 0))],
        out_specs=pl.BlockSpec((OUT_PIX, 64, 256), lambda nb, cb: (0, cb, nb)),
        out_shape=jax.ShapeDtypeStruct((OUT_PIX, C0 + C1, N_ROIS), jnp.float32),
    )(v)
    return jnp.transpose(res.reshape(7, 7, C0 + C1, N_ROIS), (3, 2, 0, 1))


def _bcast(vec, lane):
    return vec.at[jnp.full((L,), lane, jnp.int32)].get(
        mode="promise_in_bounds")


def _sc_body(t0, t1, recs_h, out,
             buf0a, buf0b, buf1, out_t0, out_t1, idxA0, idxA1, idxB,
             wA0, wA1, wB, recs_v,
             semA0, semA1, semB, semO0, semO1):
    wid = lax.axis_index("s") * NC + lax.axis_index("c")
    base = wid * ROIS_PER_W
    pltpu.sync_copy(recs_h.at[pl.ds(base * 16, 16 * ROIS_PER_W)], recs_v)
    iota = lax.iota(jnp.int32, L)
    posc = [iota * OUT_PIX + k * (16 * OUT_PIX) for k in range(C1 // 16)]
    zeros = iota * 0
    for idxR in (idxA0, idxA1, idxB):
        for s in range(6):
            plsc.store_scatter(idxR, [iota + 16 * s], zeros)
        plsc.store_scatter(idxR, [iota + 88], zeros)

    def build_half(lv, p0, idxR, wR):
        sx1, sy1, dx, dy, bbase, HW = lv
        fmax = float(HW - 1)
        for c in range(2):
            q = iota + c * 16
            p = jnp.minimum(q + p0, 48)
            i = lax.shift_right_logical(p * 9363, 16)
            j = p - i * 7
            ty = i.astype(jnp.float32) * (1.0 / 6.0)
            tx = j.astype(jnp.float32) * (1.0 / 6.0)
            ys = sy1 + dy * ty
            xs = sx1 + dx * tx
            ys = jnp.minimum(jnp.maximum(ys, 0.0), fmax)
            xs = jnp.minimum(jnp.maximum(xs, 0.0), fmax)
            y0i = ys.astype(jnp.int32)
            x0i = xs.astype(jnp.int32)
            wy = ys - y0i.astype(jnp.float32)
            wx = xs - x0i.astype(jnp.float32)
            y1i = jnp.minimum(y0i + 1, HW - 1)
            x1i = jnp.minimum(x0i + 1, HW - 1)
            ry0 = bbase + y0i * HW
            ry1 = bbase + y1i * HW
            msk = q < 25
            qp = q * 4
            plsc.store_scatter(idxR, [qp], ry0 + x0i, mask=msk)
            plsc.store_scatter(idxR, [qp + 1], ry0 + x1i, mask=msk)
            plsc.store_scatter(idxR, [qp + 2], ry1 + x0i, mask=msk)
            plsc.store_scatter(idxR, [qp + 3], ry1 + x1i, mask=msk)
            omy = 1.0 - wy
            omx = 1.0 - wx
            sl = pl.ds(c * 16, 16)
            wR[0, sl] = omy * omx
            wR[1, sl] = omy * wx
            wR[2, sl] = wy * omx
            wR[3, sl] = wy * wx

    def compute_half(buf, wR, out_t, p0, npix, C):
        wv = [[wR[c, pl.ds(ch * 16, 16)] for c in range(4)] for ch in range(2)]

        def make_body(ws, off):
            def q_body(q, carry2=0):
                a = _bcast(ws[0], q - off)
                b = _bcast(ws[1], q - off)
                cc = _bcast(ws[2], q - off)
                d = _bcast(ws[3], q - off)
                p = p0 + q
                r0 = 4 * q
                for k in range(C // 16):
                    sl = pl.ds(k * 16, 16)
                    v00 = buf[r0, sl]
                    v01 = buf[r0 + 1, sl]
                    v10 = buf[r0 + 2, sl]
                    v11 = buf[r0 + 3, sl]
                    acc = a * v00 + b * v01 + cc * v10 + d * v11
                    plsc.store_scatter(out_t, [posc[k] + p], acc)
                return carry2
            return q_body

        lax.fori_loop(0, 16, make_body(wv[0], 0), 0)
        lax.fori_loop(16, npix, make_body(wv[1], 16), 0)

    def flush(n, out_t, sz, out_off, sem):
        return pltpu.make_async_copy(
            out_t.at[pl.ds(0, sz)],
            out.at[pl.ds(n * OUT_COLS + out_off, sz)], sem)

    SZ0 = C0 * OUT_PIX
    SZ1 = C1 * OUT_PIX

    def roi_body(r, carry):
        n = base + r
        rec = recs_v[pl.ds(16 * r, 16)]
        x1 = _bcast(rec, 0)
        y1 = _bcast(rec, 1)
        x2 = _bcast(rec, 2)
        y2 = _bcast(rec, 3)
        bI = _bcast(rec, 4).astype(jnp.int32)
        lv0 = (x1 * 0.125, y1 * 0.125, (x2 - x1) * 0.125, (y2 - y1) * 0.125,
               bI * (H0 * H0), H0)
        lv1 = (x1 * 0.0625, y1 * 0.0625, (x2 - x1) * 0.0625,
               (y2 - y1) * 0.0625, bI * (H1 * H1), H1)

        build_half(lv0, 0, idxA0, wA0)
        g0 = pltpu.async_copy(t0.at[idxA0], buf0a, semA0)
        build_half(lv0, 25, idxA1, wA1)
        g1 = pltpu.async_copy(t0.at[idxA1], buf0b, semA1)
        build_half(lv1, 0, idxB, wB)
        g2 = pltpu.async_copy(t1.at[idxB], buf1, semB)
        g0.wait()

        @pl.when(r > 0)
        def _():
            flush(n - 1, out_t0, SZ0, 0, semO0).wait()

        compute_half(buf0a, wA0, out_t0, 0, 25, C0)
        g1.wait()
        compute_half(buf0b, wA1, out_t0, 25, 24, C0)
        flush(n, out_t0, SZ0, 0, semO0).start()
        g2.wait()

        @pl.when(r > 0)
        def _():
            flush(n - 1, out_t1, SZ1, OFF1, semO1).wait()

        compute_half(buf1, wB, out_t1, 0, 25, C1)
        build_half(lv1, 25, idxB, wB)
        g3 = pltpu.async_copy(t1.at[idxB], buf1, semB)
        g3.wait()
        compute_half(buf1, wB, out_t1, 25, 24, C1)
        flush(n, out_t1, SZ1, OFF1, semO1).start()
        return carry

    lax.fori_loop(0, ROIS_PER_W, roi_body, 0)
    last = base + ROIS_PER_W - 1
    flush(last, out_t0, SZ0, 0, semO0).wait()
    flush(last, out_t1, SZ1, OFF1, semO1).wait()


@functools.partial(
    pl.kernel,
    out_type=jax.ShapeDtypeStruct((N_ROIS * OUT_COLS,), jnp.float32),
    mesh=plsc.VectorSubcoreMesh(core_axis_name="c", subcore_axis_name="s"),
    compiler_params=pltpu.CompilerParams(needs_layout_passes=False),
    scratch_types=[
        pltpu.VMEM((104, CR0), jnp.float32),
        pltpu.VMEM((104, CR0), jnp.float32),
        pltpu.VMEM((104, CR1), jnp.float32),
        pltpu.VMEM((C0 * OUT_PIX,), jnp.float32),
        pltpu.VMEM((C1 * OUT_PIX,), jnp.float32),
        pltpu.VMEM((104,), jnp.int32),
        pltpu.VMEM((104,), jnp.int32),
        pltpu.VMEM((104,), jnp.int32),
        pltpu.VMEM((4, 32), jnp.float32),
        pltpu.VMEM((4, 32), jnp.float32),
        pltpu.VMEM((4, 32), jnp.float32),
        pltpu.VMEM((16 * ROIS_PER_W,), jnp.float32),
        pltpu.SemaphoreType.DMA,
        pltpu.SemaphoreType.DMA,
        pltpu.SemaphoreType.DMA,
        pltpu.SemaphoreType.DMA,
        pltpu.SemaphoreType.DMA,
    ],
)
def _sc_call(t0, t1, recs_h, out, *scratch):
    _sc_body(t0, t1, recs_h, out, *scratch)


def kernel(feat0, feat1, rois, roi_inds):
    t0, t1 = _to_tables(feat0, feat1)
    recs = jnp.concatenate(
        [rois, roi_inds[:, None].astype(jnp.float32),
         jnp.zeros((N_ROIS, 11), jnp.float32)], axis=1)
    out = _sc_call(t0, t1, recs.reshape(-1))
    return _to_output(out)


# speedup vs baseline: 1.0548x; 1.0548x over previous
import functools

import jax
import jax.numpy as jnp
from jax import lax
from jax.experimental import pallas as pl
from jax.experimental.pallas import tpu as pltpu
from jax.experimental.pallas import tpu_sc as plsc

L = 16
N_ROIS = 512
NC, NS = 2, 16
NW = NC * NS
ROIS_PER_W = N_ROIS // NW
C0, H0, CR0 = 192, 64, 256
C1, H1, CR1 = 384, 32, 384
OUT_PIX = 49
OFF1 = C0 * OUT_PIX
OUT_COLS = (C0 + C1) * OUT_PIX


def _tp(x_ref, o_ref):
    t = jnp.transpose(x_ref[...], (0, 2, 3, 1))
    pad = o_ref.shape[-1] - t.shape[-1]
    if pad:
        t = jnp.concatenate(
            [t, jnp.zeros(t.shape[:-1] + (pad,), t.dtype)], axis=-1)
    o_ref[...] = t


def _tables_body(x0_ref, x1_ref, o0_ref, o1_ref):
    _tp(x0_ref, o0_ref)
    _tp(x1_ref, o1_ref)


def _to_tables(feat0, feat1):
    t0, t1 = pl.pallas_call(
        _tables_body,
        grid=(2, 2),
        in_specs=[pl.BlockSpec((1, C0, H0 // 2, H0), lambda b, h: (b, 0, h, 0)),
                  pl.BlockSpec((1, C1, H1 // 2, H1), lambda b, h: (b, 0, h, 0))],
        out_specs=[pl.BlockSpec((1, H0 // 2, H0, CR0),
                                lambda b, h: (b, h, 0, 0)),
                   pl.BlockSpec((1, H1 // 2, H1, CR1),
                                lambda b, h: (b, h, 0, 0))],
        out_shape=[jax.ShapeDtypeStruct((2, H0, H0, CR0), jnp.float32),
                   jax.ShapeDtypeStruct((2, H1, H1, CR1), jnp.float32)],
    )(feat0, feat1)
    return t0.reshape(2 * H0 * H0, CR0), t1.reshape(2 * H1 * H1, CR1)


def _relayout_body(x_ref, o_ref):
    o_ref[...] = jnp.transpose(x_ref[...], (2, 1, 0))


def _to_output(flat):
    v = flat.reshape(N_ROIS, C0 + C1, OUT_PIX)
    res = pl.pallas_call(
        _relayout_body,
        grid=(2, 9),
        in_specs=[pl.BlockSpec((256, 64, OUT_PIX), lambda nb, cb: (nb, cb, 0))],
        out_specs=pl.BlockSpec((OUT_PIX, 64, 256), lambda nb, cb: (0, cb, nb)),
        out_shape=jax.ShapeDtypeStruct((OUT_PIX, C0 + C1, N_ROIS), jnp.float32),
    )(v)
    return jnp.transpose(res.reshape(7, 7, C0 + C1, N_ROIS), (3, 2, 0, 1))


def _bcast(vec, lane):
    return vec.at[jnp.full((L,), lane, jnp.int32)].get(
        mode="promise_in_bounds")


def _sc_body(t0, t1, recs_h, out,
             buf0, buf1, out_t0, out_t1, idxA, idxB, wA, wB, recs_v,
             semA, semB, semO0, semO1):
    wid = lax.axis_index("s") * NC + lax.axis_index("c")
    base = wid * ROIS_PER_W
    pltpu.sync_copy(recs_h.at[pl.ds(base * 16, 16 * ROIS_PER_W)], recs_v)
    iota = lax.iota(jnp.int32, L)
    posc = [iota * OUT_PIX + k * (16 * OUT_PIX) for k in range(C1 // 16)]
    zeros = iota * 0
    for idxR in (idxA, idxB):
        for s in range(6):
            plsc.store_scatter(idxR, [iota + 16 * s], zeros)
        plsc.store_scatter(idxR, [iota + 88], zeros)

    def build_half(lv, p0, idxR, wR):
        sx1, sy1, dx, dy, bbase, HW = lv
        fmax = float(HW - 1)
        for c in range(2):
            q = iota + c * 16
            p = jnp.minimum(q + p0, 48)
            i = lax.shift_right_logical(p * 9363, 16)
            j = p - i * 7
            ty = i.astype(jnp.float32) * (1.0 / 6.0)
            tx = j.astype(jnp.float32) * (1.0 / 6.0)
            ys = sy1 + dy * ty
            xs = sx1 + dx * tx
            ys = jnp.minimum(jnp.maximum(ys, 0.0), fmax)
            xs = jnp.minimum(jnp.maximum(xs, 0.0), fmax)
            y0i = ys.astype(jnp.int32)
            x0i = xs.astype(jnp.int32)
            wy = ys - y0i.astype(jnp.float32)
            wx = xs - x0i.astype(jnp.float32)
            y1i = jnp.minimum(y0i + 1, HW - 1)
            x1i = jnp.minimum(x0i + 1, HW - 1)
            ry0 = bbase + y0i * HW
            ry1 = bbase + y1i * HW
            msk = q < 25
            qp = q * 4
            plsc.store_scatter(idxR, [qp], ry0 + x0i, mask=msk)
            plsc.store_scatter(idxR, [qp + 1], ry0 + x1i, mask=msk)
            plsc.store_scatter(idxR, [qp + 2], ry1 + x0i, mask=msk)
            plsc.store_scatter(idxR, [qp + 3], ry1 + x1i, mask=msk)
            omy = 1.0 - wy
            omx = 1.0 - wx
            sl = pl.ds(c * 16, 16)
            wR[0, sl] = omy * omx
            wR[1, sl] = omy * wx
            wR[2, sl] = wy * omx
            wR[3, sl] = wy * wx

    def compute_half(buf, wR, out_t, p0, npix, C):
        wv = [[wR[c, pl.ds(ch * 16, 16)] for c in range(4)] for ch in range(2)]

        def make_body(ws, off):
            def q_body(q, carry2=0):
                a = _bcast(ws[0], q - off)
                b = _bcast(ws[1], q - off)
                cc = _bcast(ws[2], q - off)
                d = _bcast(ws[3], q - off)
                p = p0 + q
                r0 = 4 * q
                for k in range(C // 16):
                    sl = pl.ds(k * 16, 16)
                    v00 = buf[r0, sl]
                    v01 = buf[r0 + 1, sl]
                    v10 = buf[r0 + 2, sl]
                    v11 = buf[r0 + 3, sl]
                    acc = a * v00 + b * v01 + cc * v10 + d * v11
                    plsc.store_scatter(out_t, [posc[k] + p], acc)
                return carry2
            return q_body

        lax.fori_loop(0, 16, make_body(wv[0], 0), 0)
        lax.fori_loop(16, npix, make_body(wv[1], 16), 0)

    def flush(n, out_t, sz, out_off, sem):
        return pltpu.make_async_copy(
            out_t.at[pl.ds(0, sz)],
            out.at[pl.ds(n * OUT_COLS + out_off, sz)], sem)

    SZ0 = C0 * OUT_PIX
    SZ1 = C1 * OUT_PIX

    def roi_body(r, carry):
        n = base + r
        rec = recs_v[pl.ds(16 * r, 16)]
        x1 = _bcast(rec, 0)
        y1 = _bcast(rec, 1)
        x2 = _bcast(rec, 2)
        y2 = _bcast(rec, 3)
        bI = _bcast(rec, 4).astype(jnp.int32)
        lv0 = (x1 * 0.125, y1 * 0.125, (x2 - x1) * 0.125, (y2 - y1) * 0.125,
               bI * (H0 * H0), H0)
        lv1 = (x1 * 0.0625, y1 * 0.0625, (x2 - x1) * 0.0625,
               (y2 - y1) * 0.0625, bI * (H1 * H1), H1)

        build_half(lv0, 0, idxA, wA)
        gA = pltpu.async_copy(t0.at[idxA], buf0, semA)
        build_half(lv1, 0, idxB, wB)
        gB = pltpu.async_copy(t1.at[idxB], buf1, semB)
        gA.wait()

        @pl.when(r > 0)
        def _():
            flush(n - 1, out_t0, SZ0, 0, semO0).wait()

        compute_half(buf0, wA, out_t0, 0, 25, C0)
        build_half(lv0, 25, idxA, wA)
        gA2 = pltpu.async_copy(t0.at[idxA], buf0, semA)
        gB.wait()

        @pl.when(r > 0)
        def _():
            flush(n - 1, out_t1, SZ1, OFF1, semO1).wait()

        compute_half(buf1, wB, out_t1, 0, 25, C1)
        build_half(lv1, 25, idxB, wB)
        gB2 = pltpu.async_copy(t1.at[idxB], buf1, semB)
        gA2.wait()
        compute_half(buf0, wA, out_t0, 25, 24, C0)
        flush(n, out_t0, SZ0, 0, semO0).start()
        gB2.wait()
        compute_half(buf1, wB, out_t1, 25, 24, C1)
        flush(n, out_t1, SZ1, OFF1, semO1).start()
        return carry

    lax.fori_loop(0, ROIS_PER_W, roi_body, 0)
    last = base + ROIS_PER_W - 1
    flush(last, out_t0, SZ0, 0, semO0).wait()
    flush(last, out_t1, SZ1, OFF1, semO1).wait()


@functools.partial(
    pl.kernel,
    out_type=jax.ShapeDtypeStruct((N_ROIS * OUT_COLS,), jnp.float32),
    mesh=plsc.VectorSubcoreMesh(core_axis_name="c", subcore_axis_name="s"),
    compiler_params=pltpu.CompilerParams(needs_layout_passes=False),
    scratch_types=[
        pltpu.VMEM((104, CR0), jnp.float32),
        pltpu.VMEM((104, CR1), jnp.float32),
        pltpu.VMEM((C0 * OUT_PIX,), jnp.float32),
        pltpu.VMEM((C1 * OUT_PIX,), jnp.float32),
        pltpu.VMEM((104,), jnp.int32),
        pltpu.VMEM((104,), jnp.int32),
        pltpu.VMEM((4, 32), jnp.float32),
        pltpu.VMEM((4, 32), jnp.float32),
        pltpu.VMEM((16 * ROIS_PER_W,), jnp.float32),
        pltpu.SemaphoreType.DMA,
        pltpu.SemaphoreType.DMA,
        pltpu.SemaphoreType.DMA,
        pltpu.SemaphoreType.DMA,
    ],
)
def _sc_call(t0, t1, recs_h, out, *scratch):
    _sc_body(t0, t1, recs_h, out, *scratch)


def kernel(feat0, feat1, rois, roi_inds):
    t0, t1 = _to_tables(feat0, feat1)
    recs = jnp.concatenate(
        [rois, roi_inds[:, None].astype(jnp.float32),
         jnp.zeros((N_ROIS, 11), jnp.float32)], axis=1)
    out = _sc_call(t0, t1, recs.reshape(-1))
    return _to_output(out)
